# speedup vs baseline: 1.2549x; 1.2549x over previous
import functools

import jax
import jax.numpy as jnp
from jax import lax
from jax.experimental import pallas as pl
from jax.experimental.pallas import tpu as pltpu
from jax.experimental.pallas import tpu_sc as plsc

VOCAB = 50368
HIDDEN = 768
EPS = 1e-05
BATCH = 4
SEQ = 8192

NUM_TOKENS = BATCH * SEQ
NC = 2
NS = 16
NW = NC * NS
B_PER_W = NUM_TOKENS // NW
CHUNK = 64
N_CHUNKS = B_PER_W // CHUNK


def _sc_gather(table, idx_flat):
    mesh = plsc.VectorSubcoreMesh(core_axis_name="c", subcore_axis_name="s")

    @functools.partial(
        pl.kernel,
        out_type=jax.ShapeDtypeStruct((NUM_TOKENS, HIDDEN), jnp.float32),
        mesh=mesh,
        scratch_types=[
            pltpu.VMEM((CHUNK,), jnp.int32),
            pltpu.VMEM((CHUNK, HIDDEN), jnp.float32),
            pltpu.SemaphoreType.DMA,
        ],
    )
    def gather_kernel(table_hbm, idx_hbm, out_hbm, idx_v, rows_v, sem):
        wid = lax.axis_index("s") * NC + lax.axis_index("c")
        base = wid * B_PER_W

        @pl.loop(0, N_CHUNKS)
        def _(c):
            off = base + c * CHUNK
            pltpu.sync_copy(idx_hbm.at[pl.ds(off, CHUNK)], idx_v)
            pltpu.async_copy(table_hbm.at[idx_v], rows_v, sem).wait()
            pltpu.sync_copy(rows_v, out_hbm.at[pl.ds(off, CHUNK)])

    return gather_kernel(table, idx_flat)


_LN_BLOCK = 1024


def _ln_body(x_ref, w_ref, o_ref):
    x = x_ref[...]
    mean = jnp.mean(x, axis=1, keepdims=True)
    xc = x - mean
    var = jnp.mean(xc * xc, axis=1, keepdims=True)
    o_ref[...] = xc * lax.rsqrt(var + EPS) * w_ref[...]


def _tc_layernorm(x, w):
    n = x.shape[0]
    return pl.pallas_call(
        _ln_body,
        grid=(n // _LN_BLOCK,),
        in_specs=[
            pl.BlockSpec((_LN_BLOCK, HIDDEN), lambda i: (i, 0)),
            pl.BlockSpec((1, HIDDEN), lambda i: (0, 0)),
        ],
        out_specs=pl.BlockSpec((_LN_BLOCK, HIDDEN), lambda i: (i, 0)),
        out_shape=jax.ShapeDtypeStruct((n, HIDDEN), jnp.float32),
    )(x, w.reshape(1, HIDDEN))


def kernel(input_ids, tok_embeddings, norm_weight):
    idx_flat = input_ids.reshape(NUM_TOKENS)
    gathered = _sc_gather(tok_embeddings, idx_flat)
    normed = _tc_layernorm(gathered, norm_weight)
    return normed.reshape(BATCH, SEQ, HIDDEN)


# speedup vs baseline: 1.3234x; 1.0546x over previous
import functools

import jax
import jax.numpy as jnp
from jax import lax
from jax.experimental import pallas as pl
from jax.experimental.pallas import tpu as pltpu
from jax.experimental.pallas import tpu_sc as plsc

VOCAB = 50368
HIDDEN = 768
EPS = 1e-05
BATCH = 4
SEQ = 8192

NUM_TOKENS = BATCH * SEQ
NC = 2
NS = 16
NW = NC * NS
GROUPS = 4
GROUP_TOKENS = NUM_TOKENS // GROUPS
B_PER_W = GROUP_TOKENS // NW
CHUNK = 64
N_CHUNKS = B_PER_W // CHUNK


def _sc_gather_group(table, idx_group):
    mesh = plsc.VectorSubcoreMesh(core_axis_name="c", subcore_axis_name="s")

    @functools.partial(
        pl.kernel,
        out_type=jax.ShapeDtypeStruct((GROUP_TOKENS, HIDDEN), jnp.float32),
        mesh=mesh,
        scratch_types=[
            pltpu.VMEM((B_PER_W,), jnp.int32),
            pltpu.VMEM((CHUNK, HIDDEN), jnp.float32),
            pltpu.VMEM((CHUNK, HIDDEN), jnp.float32),
            pltpu.SemaphoreType.DMA,
            pltpu.SemaphoreType.DMA,
        ],
    )
    def gather_kernel(table_hbm, idx_hbm, out_hbm, idx_v, rows_a, rows_b, sem_a, sem_b):
        wid = lax.axis_index("s") * NC + lax.axis_index("c")
        base = wid * B_PER_W
        pltpu.sync_copy(idx_hbm.at[pl.ds(base, B_PER_W)], idx_v)

        bufs = (rows_a, rows_b)
        sems = (sem_a, sem_b)

        def start(c):
            pltpu.async_copy(
                table_hbm.at[idx_v.at[pl.ds(c * CHUNK, CHUNK)]],
                bufs[c % 2], sems[c % 2])

        start(0)
        for c in range(N_CHUNKS):
            pltpu.make_async_copy(
                table_hbm.at[idx_v.at[pl.ds(c * CHUNK, CHUNK)]],
                bufs[c % 2], sems[c % 2]).wait()
            if c + 1 < N_CHUNKS:
                start(c + 1)
            pltpu.sync_copy(bufs[c % 2],
                            out_hbm.at[pl.ds(base + c * CHUNK, CHUNK)])

    return gather_kernel(table, idx_group)


_LN_BLOCK = 1024
_BLOCKS_PER_GROUP = GROUP_TOKENS // _LN_BLOCK


def _ln_body_first(x_ref, w_ref, o_ref):
    x = x_ref[...]
    mean = jnp.mean(x, axis=1, keepdims=True)
    xc = x - mean
    var = jnp.mean(xc * xc, axis=1, keepdims=True)
    o_ref[...] = xc * lax.rsqrt(var + EPS) * w_ref[...]


def _ln_body_chain(x_ref, w_ref, buf_ref, o_ref):
    del buf_ref
    _ln_body_first(x_ref, w_ref, o_ref)


def _tc_layernorm_group(g, x_group, w2d, buf):
    out_shape = jax.ShapeDtypeStruct((NUM_TOKENS, HIDDEN), jnp.float32)
    out_spec = pl.BlockSpec(
        (_LN_BLOCK, HIDDEN), lambda i, g=g: (g * _BLOCKS_PER_GROUP + i, 0))
    x_spec = pl.BlockSpec((_LN_BLOCK, HIDDEN), lambda i: (i, 0))
    w_spec = pl.BlockSpec((1, HIDDEN), lambda i: (0, 0))
    if g == 0:
        return pl.pallas_call(
            _ln_body_first,
            grid=(_BLOCKS_PER_GROUP,),
            in_specs=[x_spec, w_spec],
            out_specs=out_spec,
            out_shape=out_shape,
        )(x_group, w2d)
    return pl.pallas_call(
        _ln_body_chain,
        grid=(_BLOCKS_PER_GROUP,),
        in_specs=[x_spec, w_spec,
                  pl.BlockSpec(memory_space=pltpu.MemorySpace.HBM)],
        out_specs=out_spec,
        out_shape=out_shape,
        input_output_aliases={2: 0},
    )(x_group, w2d, buf)


def kernel(input_ids, tok_embeddings, norm_weight):
    idx_flat = input_ids.reshape(NUM_TOKENS)
    w2d = norm_weight.reshape(1, HIDDEN)
    gathered = [
        _sc_gather_group(tok_embeddings,
                         lax.slice(idx_flat, (g * GROUP_TOKENS,),
                                   ((g + 1) * GROUP_TOKENS,)))
        for g in range(GROUPS)
    ]
    buf = None
    for g in range(GROUPS):
        buf = _tc_layernorm_group(g, gathered[g], w2d, buf)
    return buf.reshape(BATCH, SEQ, HIDDEN)
